# Initial kernel scaffold; baseline (speedup 1.0000x reference)
import jax
import jax.numpy as jnp
from jax.experimental import pallas as pl


def kernel(x, codebook):
    raise NotImplementedError("write your pallas kernel here")


# speedup vs baseline: 1.0259x; 1.0259x over previous
import functools

import jax
import jax.numpy as jnp
from jax import lax
from jax.experimental import pallas as pl
from jax.experimental.pallas import tpu as pltpu
from jax.experimental.pallas import tpu_sc as plsc


def _argmin_body(x_ref, cb_ref, idx_ref, *, kb, ksteps):
    x = x_ref[...]
    x2 = jnp.sum(x * x, axis=1, keepdims=True)
    nb = x.shape[0]

    def step(j, carry):
        minval, minidx = carry
        cb = cb_ref[pl.ds(j * kb, kb), :]
        c2 = jnp.sum(cb * cb, axis=1)
        cross = lax.dot_general(
            x.astype(jnp.bfloat16), cb.astype(jnp.bfloat16),
            (((1,), (1,)), ((), ())),
            preferred_element_type=jnp.float32)
        sq = (x2 + c2[None, :]) - 2.0 * cross
        d = jnp.sqrt(jnp.maximum(sq, 0.0))
        dmin = jnp.min(d, axis=1)
        kiota = lax.broadcasted_iota(jnp.int32, d.shape, 1)
        cand = jnp.where(d == dmin[:, None], kiota, jnp.int32(2**30))
        darg = jnp.min(cand, axis=1) + j * kb
        upd = dmin < minval
        dkeep = dmin.astype(jnp.bfloat16).astype(jnp.float32)
        return (jnp.where(upd, dkeep, minval),
                jnp.where(upd, darg, minidx))

    init = (jnp.full((nb,), jnp.inf, jnp.float32),
            jnp.zeros((nb,), jnp.int32))
    _, minidx = lax.fori_loop(0, ksteps, step, init)
    idx_ref[...] = minidx


def _nearest_indices(xf, codebook, *, nb=512, kb=2048):
    n, d = xf.shape
    k, _ = codebook.shape
    grid = (n // nb,)
    return pl.pallas_call(
        functools.partial(_argmin_body, kb=kb, ksteps=k // kb),
        grid=grid,
        in_specs=[
            pl.BlockSpec((nb, d), lambda i: (i, 0)),
            pl.BlockSpec((k, d), lambda i: (0, 0)),
        ],
        out_specs=pl.BlockSpec((nb,), lambda i: (i,)),
        out_shape=jax.ShapeDtypeStruct((n,), jnp.int32),
    )(xf, codebook)


def _sc_gather(codebook, idx):
    n = idx.shape[0]
    k, d = codebook.shape
    info = plsc.get_sparse_core_info()
    nw = info.num_cores * info.num_subcores
    b_per_w = n // nw
    chunk = 128
    nchunks = b_per_w // chunk
    mesh = plsc.VectorSubcoreMesh(core_axis_name="c", subcore_axis_name="s")

    @functools.partial(
        pl.kernel,
        out_type=jax.ShapeDtypeStruct((n, d), jnp.float32),
        mesh=mesh,
        scratch_types=[
            pltpu.VMEM((nchunks, chunk), jnp.int32),
            pltpu.VMEM((b_per_w, d), jnp.float32),
            pltpu.SemaphoreType.DMA,
        ],
        compiler_params=pltpu.CompilerParams(use_tc_tiling_on_sc=False),
    )
    def gather_kernel(cb_hbm, idx_hbm, out_hbm, idx_v, rows_v, sem):
        wid = lax.axis_index("s") * info.num_cores + lax.axis_index("c")
        base = wid * b_per_w
        pltpu.sync_copy(idx_hbm.at[wid], idx_v)
        copies = []
        for j in range(nchunks):
            copies.append(pltpu.async_copy(
                cb_hbm.at[idx_v.at[j]],
                rows_v.at[pl.ds(j * chunk, chunk), :],
                sem))
        for c in copies:
            c.wait()
        pltpu.sync_copy(rows_v, out_hbm.at[pl.ds(base, b_per_w)])

    return gather_kernel(codebook, idx.reshape(nw, nchunks, chunk))


def kernel(x, codebook):
    b, n, d = x.shape
    xf = x.reshape(b * n, d)
    idx = _nearest_indices(xf, codebook)
    quantized = _sc_gather(codebook, idx)
    return idx.reshape(b, n), quantized.reshape(b, n, d)
